# baseline (speedup 1.0000x reference)
import functools

import jax
import jax.numpy as jnp
from jax import lax
from jax.experimental import pallas as pl
from jax.experimental.pallas import tpu as pltpu
from jax.experimental.pallas import tpu_sc as plsc

NUM_FIELDS = 26
FIELD_SIZE = 100000
BATCH = 16384
CHUNK = 8192
LANES = 16
VECS = CHUNK // LANES

_mesh = plsc.VectorSubcoreMesh(core_axis_name="c", subcore_axis_name="s")


@functools.partial(
    pl.kernel,
    out_type=jax.ShapeDtypeStruct((NUM_FIELDS, BATCH), jnp.float32),
    mesh=_mesh,
    scratch_types=[
        pltpu.VMEM((FIELD_SIZE,), jnp.float32),
        pltpu.VMEM((CHUNK,), jnp.int32),
        pltpu.VMEM((CHUNK,), jnp.float32),
    ],
    compiler_params=pltpu.CompilerParams(needs_layout_passes=False),
)
def _gather_fields(w_hbm, xt_hbm, out_hbm, table_v, x_v, emb_v):
    f = lax.axis_index("c") * 16 + lax.axis_index("s")

    @pl.when(f < NUM_FIELDS)
    def _():
        pltpu.sync_copy(w_hbm.at[pl.ds(f * FIELD_SIZE, FIELD_SIZE)], table_v)

        def chunk_body(c, carry):
            pltpu.sync_copy(xt_hbm.at[f, pl.ds(c * CHUNK, CHUNK)], x_v)

            def vec_body(i, carry2):
                idx = x_v[pl.ds(i * LANES, LANES)]
                emb_v[pl.ds(i * LANES, LANES)] = plsc.load_gather(
                    table_v, [idx]
                )
                return carry2

            lax.fori_loop(0, VECS, vec_body, 0, unroll=4)
            pltpu.sync_copy(emb_v, out_hbm.at[f, pl.ds(c * CHUNK, CHUNK)])
            return carry

        lax.fori_loop(0, BATCH // CHUNK, chunk_body, 0)


def _reduce_body(p_ref, b_ref, o_ref):
    o_ref[...] = jnp.sum(p_ref[...], axis=0, keepdims=True) + b_ref[0, 0]


@jax.jit
def kernel(x, W, bias):
    xt = x.T
    w_flat = W.reshape(-1)
    partials = _gather_fields(w_flat, xt)
    return partials
    out = pl.pallas_call(
        _reduce_body,
        out_shape=jax.ShapeDtypeStruct((1, BATCH), jnp.float32),
    )(partials, bias.reshape(1, 1))
    return out.reshape(BATCH, 1)


# speedup vs baseline: 3.6859x; 3.6859x over previous
import functools

import jax
import jax.numpy as jnp
from jax import lax
from jax.experimental import pallas as pl
from jax.experimental.pallas import tpu as pltpu
from jax.experimental.pallas import tpu_sc as plsc

NUM_FIELDS = 26
FIELD_SIZE = 100000
BATCH = 16384
CHUNK = 8192
LANES = 16
VECS = CHUNK // LANES
STAGE = FIELD_SIZE + 96

_mesh = plsc.VectorSubcoreMesh(core_axis_name="c", subcore_axis_name="s")


@functools.partial(
    pl.kernel,
    out_type=jax.ShapeDtypeStruct((NUM_FIELDS, BATCH), jnp.float32),
    mesh=_mesh,
    scratch_types=[
        pltpu.VMEM((STAGE,), jnp.float32),
        pltpu.VMEM((CHUNK,), jnp.int32),
        pltpu.VMEM((CHUNK,), jnp.float32),
    ],
    compiler_params=pltpu.CompilerParams(needs_layout_passes=False),
)
def _gather_fields(w_hbm, xt_hbm, out_hbm, table_v, x_v, emb_v):
    f = lax.axis_index("c") * 16 + lax.axis_index("s")

    @pl.when(f < NUM_FIELDS)
    def _():
        base = f * FIELD_SIZE
        corr = lax.rem(base, 128)
        aligned = pl.multiple_of(base - corr, 128)
        pltpu.sync_copy(w_hbm.at[0, pl.ds(aligned, STAGE)], table_v)

        def chunk_body(c, carry):
            pltpu.sync_copy(xt_hbm.at[f, pl.ds(c * CHUNK, CHUNK)], x_v)

            def vec_body(i, carry2):
                idx = x_v[pl.ds(i * LANES, LANES)] + corr
                emb_v[pl.ds(i * LANES, LANES)] = plsc.load_gather(
                    table_v, [idx]
                )
                return carry2

            lax.fori_loop(0, VECS, vec_body, 0, unroll=4)
            pltpu.sync_copy(emb_v, out_hbm.at[f, pl.ds(c * CHUNK, CHUNK)])
            return carry

        lax.fori_loop(0, BATCH // CHUNK, chunk_body, 0)


def _reduce_body(p_ref, b_ref, o_ref):
    o_ref[...] = jnp.sum(p_ref[...], axis=0, keepdims=True) + b_ref[0, 0]


@jax.jit
def kernel(x, W, bias):
    xt = x.T
    partials = _gather_fields(W.T, xt)
    out = pl.pallas_call(
        _reduce_body,
        out_shape=jax.ShapeDtypeStruct((1, BATCH), jnp.float32),
    )(partials, bias.reshape(1, 1))
    return out.reshape(BATCH, 1)


# speedup vs baseline: 4.9341x; 1.3386x over previous
import functools

import jax
import jax.numpy as jnp
from jax import lax
from jax.experimental import pallas as pl
from jax.experimental.pallas import tpu as pltpu
from jax.experimental.pallas import tpu_sc as plsc

NUM_FIELDS = 26
FIELD_SIZE = 100000
BATCH = 16384
CHUNK = 8192
LANES = 16
VECS = CHUNK // LANES
STAGE = FIELD_SIZE + 96

_mesh = plsc.VectorSubcoreMesh(core_axis_name="c", subcore_axis_name="s")


@functools.partial(
    pl.kernel,
    out_type=jax.ShapeDtypeStruct((NUM_FIELDS, BATCH), jnp.float32),
    mesh=_mesh,
    scratch_types=[
        pltpu.VMEM((STAGE,), jnp.float32),
        pltpu.VMEM((CHUNK,), jnp.int32),
        pltpu.VMEM((CHUNK,), jnp.int32),
        pltpu.VMEM((CHUNK,), jnp.float32),
        pltpu.SemaphoreType.DMA,
        pltpu.SemaphoreType.DMA,
    ],
    compiler_params=pltpu.CompilerParams(needs_layout_passes=False),
)
def _gather_fields(w_hbm, xt_hbm, out_hbm, table_v, x0_v, x1_v, emb_v,
                   sem_t, sem_x):
    f = lax.axis_index("c") * 16 + lax.axis_index("s")

    @pl.when(f < NUM_FIELDS)
    def _():
        base = f * FIELD_SIZE
        corr = lax.rem(base, 128)
        aligned = pl.multiple_of(base - corr, 128)
        cp_t = pltpu.make_async_copy(
            w_hbm.at[0, pl.ds(aligned, STAGE)], table_v, sem_t)
        cp_t.start()
        cp_x0 = pltpu.make_async_copy(
            xt_hbm.at[f, pl.ds(0, CHUNK)], x0_v, sem_x)
        cp_x0.start()
        cp_x1 = pltpu.make_async_copy(
            xt_hbm.at[f, pl.ds(CHUNK, CHUNK)], x1_v, sem_x)
        cp_x1.start()
        cp_t.wait()
        cp_x0.wait()

        @plsc.parallel_loop(0, VECS, 1, unroll=8)
        def _lo(i):
            idx = x0_v[pl.ds(i * LANES, LANES)] + corr
            emb_v[pl.ds(i * LANES, LANES)] = plsc.load_gather(table_v, [idx])

        pltpu.sync_copy(emb_v, out_hbm.at[f, pl.ds(0, CHUNK)])
        cp_x1.wait()

        @plsc.parallel_loop(0, VECS, 1, unroll=8)
        def _hi(i):
            idx = x1_v[pl.ds(i * LANES, LANES)] + corr
            emb_v[pl.ds(i * LANES, LANES)] = plsc.load_gather(table_v, [idx])

        pltpu.sync_copy(emb_v, out_hbm.at[f, pl.ds(CHUNK, CHUNK)])


def _reduce_body(p_ref, b_ref, o_ref):
    o_ref[...] = jnp.sum(p_ref[...], axis=0, keepdims=True) + b_ref[0, 0]


@jax.jit
def kernel(x, W, bias):
    xt = x.T
    partials = _gather_fields(W.T, xt)
    out = pl.pallas_call(
        _reduce_body,
        out_shape=jax.ShapeDtypeStruct((1, BATCH), jnp.float32),
    )(partials, bias.reshape(1, 1))
    return out.reshape(BATCH, 1)


# speedup vs baseline: 4.9347x; 1.0001x over previous
import functools

import jax
import jax.numpy as jnp
from jax import lax
from jax.experimental import pallas as pl
from jax.experimental.pallas import tpu as pltpu
from jax.experimental.pallas import tpu_sc as plsc

NUM_FIELDS = 26
FIELD_SIZE = 100000
BATCH = 16384
CHUNK = 8192
LANES = 16
VECS = CHUNK // LANES
STAGE = FIELD_SIZE + 96

_mesh = plsc.VectorSubcoreMesh(core_axis_name="c", subcore_axis_name="s")


@functools.partial(
    pl.kernel,
    out_type=jax.ShapeDtypeStruct((NUM_FIELDS, BATCH), jnp.float32),
    mesh=_mesh,
    scratch_types=[
        pltpu.VMEM((STAGE,), jnp.float32),
        pltpu.VMEM((BATCH,), jnp.int32),
        pltpu.VMEM((CHUNK,), jnp.float32),
        pltpu.SemaphoreType.DMA,
        pltpu.SemaphoreType.DMA,
    ],
    compiler_params=pltpu.CompilerParams(needs_layout_passes=False),
)
def _gather_fields(w_hbm, xt_hbm, out_hbm, table_v, x_v, emb_v,
                   sem_t, sem_x):
    f = lax.axis_index("c") * 16 + lax.axis_index("s")

    @pl.when(f < NUM_FIELDS)
    def _():
        base = f * FIELD_SIZE
        corr = lax.rem(base, 128)
        aligned = pl.multiple_of(base - corr, 128)
        cp_t = pltpu.make_async_copy(
            w_hbm.at[0, pl.ds(aligned, STAGE)], table_v, sem_t)
        cp_t.start()
        cp_x = pltpu.make_async_copy(xt_hbm.at[f, :], x_v, sem_x)
        cp_x.start()
        cp_t.wait()
        cp_x.wait()

        def chunk_body(c, carry):
            @plsc.parallel_loop(0, VECS, 1, unroll=8)
            def _gather(i):
                idx = x_v[pl.ds(c * CHUNK + i * LANES, LANES)] + corr
                emb_v[pl.ds(i * LANES, LANES)] = plsc.load_gather(
                    table_v, [idx])

            pltpu.sync_copy(emb_v, out_hbm.at[f, pl.ds(c * CHUNK, CHUNK)])
            return carry

        lax.fori_loop(0, BATCH // CHUNK, chunk_body, 0)


def _reduce_body(p_ref, b_ref, o_ref):
    o_ref[...] = jnp.sum(p_ref[...], axis=0, keepdims=True) + b_ref[0, 0]


@jax.jit
def kernel(x, W, bias):
    xt = x.T
    partials = _gather_fields(W.T, xt)
    out = pl.pallas_call(
        _reduce_body,
        out_shape=jax.ShapeDtypeStruct((1, BATCH), jnp.float32),
    )(partials, bias.reshape(1, 1))
    return out.reshape(BATCH, 1)


# speedup vs baseline: 4.9890x; 1.0110x over previous
import functools

import jax
import jax.numpy as jnp
from jax import lax
from jax.experimental import pallas as pl
from jax.experimental.pallas import tpu as pltpu
from jax.experimental.pallas import tpu_sc as plsc

NUM_FIELDS = 26
FIELDS_PER_CORE = 13
FIELD_SIZE = 100000
BATCH = 16384
CHUNK = 4096
NCHUNKS = BATCH // CHUNK
LANES = 16
VECS = CHUNK // LANES
STAGE = FIELD_SIZE + 96

_mesh = plsc.VectorSubcoreMesh(core_axis_name="c", subcore_axis_name="s")


@functools.partial(
    pl.kernel,
    out_type=jax.ShapeDtypeStruct((NUM_FIELDS, BATCH), jnp.float32),
    mesh=_mesh,
    scratch_types=[
        pltpu.VMEM((STAGE,), jnp.float32),
        pltpu.VMEM((BATCH,), jnp.int32),
        pltpu.VMEM((CHUNK,), jnp.float32),
        pltpu.VMEM((CHUNK,), jnp.float32),
        pltpu.SemaphoreType.DMA,
        pltpu.SemaphoreType.DMA,
        pltpu.SemaphoreType.DMA,
    ],
    compiler_params=pltpu.CompilerParams(needs_layout_passes=False),
)
def _gather_fields(w_hbm, xt_hbm, out_hbm, table_v, x_v, emb0_v, emb1_v,
                   sem_t, sem_x, sem_w):
    sid = lax.axis_index("s")
    f = lax.axis_index("c") * FIELDS_PER_CORE + sid

    @pl.when(sid < FIELDS_PER_CORE)
    def _():
        base = f * FIELD_SIZE
        corr = lax.rem(base, 128)
        aligned = pl.multiple_of(base - corr, 128)
        cp_t = pltpu.make_async_copy(
            w_hbm.at[0, pl.ds(aligned, STAGE)], table_v, sem_t)
        cp_t.start()
        cp_x = pltpu.make_async_copy(xt_hbm.at[f, :], x_v, sem_x)
        cp_x.start()
        cp_t.wait()
        cp_x.wait()

        embs = (emb0_v, emb1_v)
        writes = []
        for c in range(NCHUNKS):
            emb_v = embs[c % 2]
            if len(writes) >= 2:
                writes[c - 2].wait()

            @plsc.parallel_loop(0, VECS, 1, unroll=8)
            def _gather(i, c=c, emb_v=emb_v):
                idx = x_v[pl.ds(c * CHUNK + i * LANES, LANES)] + corr
                emb_v[pl.ds(i * LANES, LANES)] = plsc.load_gather(
                    table_v, [idx])

            cp_w = pltpu.make_async_copy(
                emb_v, out_hbm.at[f, pl.ds(c * CHUNK, CHUNK)], sem_w)
            cp_w.start()
            writes.append(cp_w)
        writes[-2].wait()
        writes[-1].wait()


def _reduce_body(p_ref, b_ref, o_ref):
    o_ref[...] = jnp.sum(p_ref[...], axis=0, keepdims=True) + b_ref[0, 0]


@jax.jit
def kernel(x, W, bias):
    xt = x.T
    partials = _gather_fields(W.T, xt)
    out = pl.pallas_call(
        _reduce_body,
        out_shape=jax.ShapeDtypeStruct((1, BATCH), jnp.float32),
    )(partials, bias.reshape(1, 1))
    return out.reshape(BATCH, 1)
